# baseline (speedup 1.0000x reference)
import functools

import jax
import jax.numpy as jnp
from jax import lax
from jax.experimental import pallas as pl
from jax.experimental.pallas import tpu as pltpu
from jax.experimental.pallas import tpu_sc as plsc

_NC = 2
_NS = 16
_NW = _NC * _NS
_L = 16


def _sc_scores(B, NNEG, D, center, context, neg_flat, W_center, W_context):
    b_per_w = B // _NW
    CB = 32
    NCH = b_per_w // CB
    NIDX = CB * NNEG
    NSTREAM = NIDX // 128

    mesh = plsc.VectorSubcoreMesh(core_axis_name="c", subcore_axis_name="s")

    NV = D // _L

    @functools.partial(
        pl.kernel,
        out_type=[
            jax.ShapeDtypeStruct((B,), jnp.float32),
            jax.ShapeDtypeStruct((NNEG, B), jnp.float32),
        ],
        mesh=mesh,
        scratch_types=[
            [pltpu.VMEM((CB,), jnp.int32)] * 2,
            [pltpu.VMEM((CB,), jnp.int32)] * 2,
            [pltpu.VMEM((NIDX,), jnp.int32)] * 2,
            [pltpu.VMEM((CB, D), jnp.float32)] * 2,
            [pltpu.VMEM((CB, D), jnp.float32)] * 2,
            [pltpu.VMEM((NIDX, D), jnp.float32)] * 2,
            pltpu.VMEM((B // _NW,), jnp.float32),
            pltpu.VMEM((NNEG, B // _NW), jnp.float32),
            [pltpu.SemaphoreType.DMA] * 2,
            [pltpu.SemaphoreType.DMA] * 2,
        ],
        compiler_params=pltpu.CompilerParams(needs_layout_passes=False,
                                             use_tc_tiling_on_sc=False),
    )
    def scores_kernel(center_hbm, context_hbm, neg_hbm, wc_hbm, wx_hbm,
                      pos_hbm, negt_hbm,
                      cidx_v, xidx_v, nidx_v, crows_v, xrows_v, nrows_v,
                      posb_v, negb_v, sem_idx, sem_rows):
        wid = lax.axis_index("s") * _NC + lax.axis_index("c")
        base = pl.multiple_of(wid * b_per_w, b_per_w)

        def start_idx(cb, p):
            b0 = pl.multiple_of(base + cb * CB, CB)
            pltpu.async_copy(center_hbm.at[pl.ds(b0, CB)], cidx_v[p],
                             sem_idx[p])
            pltpu.async_copy(context_hbm.at[pl.ds(b0, CB)], xidx_v[p],
                             sem_idx[p])
            pltpu.async_copy(
                neg_hbm.at[pl.ds(pl.multiple_of(b0 * NNEG, NIDX), NIDX)],
                nidx_v[p], sem_idx[p])

        def wait_idx(p):
            pltpu.make_async_copy(center_hbm.at[pl.ds(0, CB)], cidx_v[p],
                                  sem_idx[p]).wait()
            pltpu.make_async_copy(context_hbm.at[pl.ds(0, CB)], xidx_v[p],
                                  sem_idx[p]).wait()
            pltpu.make_async_copy(neg_hbm.at[pl.ds(0, NIDX)], nidx_v[p],
                                  sem_idx[p]).wait()

        def start_rows(p):
            pltpu.async_copy(wc_hbm.at[cidx_v[p]], crows_v[p], sem_rows[p])
            pltpu.async_copy(wx_hbm.at[xidx_v[p]], xrows_v[p], sem_rows[p])
            for j in range(NSTREAM):
                pltpu.async_copy(
                    wx_hbm.at[nidx_v[p].at[pl.ds(j * 128, 128)]],
                    nrows_v[p].at[pl.ds(j * 128, 128), :], sem_rows[p])

        def wait_rows(p):
            pltpu.make_async_copy(wc_hbm.at[cidx_v[p]], crows_v[p],
                                  sem_rows[p]).wait()
            pltpu.make_async_copy(wx_hbm.at[xidx_v[p]], xrows_v[p],
                                  sem_rows[p]).wait()
            for j in range(NSTREAM):
                pltpu.make_async_copy(
                    wx_hbm.at[nidx_v[p].at[pl.ds(j * 128, 128)]],
                    nrows_v[p].at[pl.ds(j * 128, 128), :],
                    sem_rows[p]).wait()

        def compute(cb, p):
            crow, xrow, nrow = crows_v[p], xrows_v[p], nrows_v[p]

            lane = lax.iota(jnp.int32, _L)
            m_last = lane == (_L - 1)

            def b_body(b, carry):
                cv = [crow[b, pl.ds(k * _L, _L)] for k in range(NV)]
                xv = [xrow[b, pl.ds(k * _L, _L)] for k in range(NV)]
                pp = ((cv[0] * xv[0] + cv[1] * xv[1])
                      + (cv[2] * xv[2] + cv[3] * xv[3]))
                idxv = jnp.full((_L,), cb * CB + b, jnp.int32)
                plsc.store_scatter(posb_v, [idxv], plsc.cumsum(pp),
                                   mask=m_last)
                nb = b * NNEG
                for n in range(NNEG):
                    nv = [nrow[nb + n, pl.ds(k * _L, _L)] for k in range(NV)]
                    np_ = ((cv[0] * nv[0] + cv[1] * nv[1])
                           + (cv[2] * nv[2] + cv[3] * nv[3]))
                    plsc.store_scatter(
                        negb_v, [jnp.full((_L,), n, jnp.int32), idxv],
                        plsc.cumsum(np_), mask=m_last)
                return carry

            lax.fori_loop(0, CB, b_body, 0)

        start_idx(0, 0)
        wait_idx(0)
        start_rows(0)
        start_idx(1, 1)

        def half_body(h, carry):
            for p in range(2):
                cb = h * 2 + p
                q = 1 - p

                @pl.when(cb + 1 < NCH)
                def _():
                    wait_idx(q)
                    start_rows(q)

                wait_rows(p)

                @pl.when(cb + 2 < NCH)
                def _():
                    start_idx(cb + 2, p)

                compute(cb, p)
            return carry

        lax.fori_loop(0, NCH // 2, half_body, 0)

        pltpu.sync_copy(posb_v, pos_hbm.at[pl.ds(base, b_per_w)])
        for n in range(NNEG):
            pltpu.sync_copy(negb_v.at[n], negt_hbm.at[n, pl.ds(base, b_per_w)])

    return scores_kernel(center, context, neg_flat, W_center, W_context)


def _loss_kernel(pos_ref, neg_ref, out_ref, *, inv_b):
    def logsig(x):
        return jnp.minimum(x, 0.0) - jnp.log1p(jnp.exp(-jnp.abs(x)))

    s_pos = jnp.sum(logsig(pos_ref[...]))
    s_neg = jnp.sum(logsig(-neg_ref[...]))
    out_ref[...] = jnp.broadcast_to(-(s_pos + s_neg) * inv_b, (1, 1))


def kernel(center, context, negatives, W_center, W_context):
    W_center, W_context = lax.optimization_barrier((W_center, W_context))
    B, NNEG = negatives.shape
    D = W_center.shape[1]
    pos, negt = _sc_scores(B, NNEG, D,
                           center.astype(jnp.int32),
                           context.astype(jnp.int32),
                           negatives.reshape(-1).astype(jnp.int32),
                           W_center, W_context)
    loss = pl.pallas_call(
        functools.partial(_loss_kernel, inv_b=1.0 / B),
        out_shape=jax.ShapeDtypeStruct((1, 1), jnp.float32),
    )(pos.reshape(B // 128, 128), negt.reshape(NNEG * B // 128, 128))
    return loss[0, 0]


# speedup vs baseline: 1.9165x; 1.9165x over previous
import functools

import jax
import jax.numpy as jnp
from jax import lax
from jax.experimental import pallas as pl
from jax.experimental.pallas import tpu as pltpu
from jax.experimental.pallas import tpu_sc as plsc

_NC = 2
_NS = 16
_NW = _NC * _NS
_L = 16


def _sc_scores(B, NNEG, D, center, context, neg_flat, W_center, W_context):
    b_per_w = B // _NW
    CB = 32
    NCH = b_per_w // CB
    NIDX = CB * NNEG
    NSTREAM = NIDX // 128

    mesh = plsc.VectorSubcoreMesh(core_axis_name="c", subcore_axis_name="s")

    NV = D // _L

    @functools.partial(
        pl.kernel,
        out_type=[
            jax.ShapeDtypeStruct((B,), jnp.float32),
            jax.ShapeDtypeStruct((NNEG, B), jnp.float32),
        ],
        mesh=mesh,
        scratch_types=[
            [pltpu.VMEM((CB,), jnp.int32)] * 2,
            [pltpu.VMEM((CB,), jnp.int32)] * 2,
            [pltpu.VMEM((NIDX,), jnp.int32)] * 2,
            [pltpu.VMEM((CB, D), jnp.float32)] * 2,
            [pltpu.VMEM((CB, D), jnp.float32)] * 2,
            [pltpu.VMEM((NIDX, D), jnp.float32)] * 2,
            pltpu.VMEM((B // _NW,), jnp.float32),
            pltpu.VMEM((NNEG, B // _NW), jnp.float32),
            [pltpu.SemaphoreType.DMA] * 2,
            [pltpu.SemaphoreType.DMA] * 2,
        ],
        compiler_params=pltpu.CompilerParams(needs_layout_passes=False,
                                             use_tc_tiling_on_sc=False),
    )
    def scores_kernel(center_hbm, context_hbm, neg_hbm, wc_hbm, wx_hbm,
                      pos_hbm, negt_hbm,
                      cidx_v, xidx_v, nidx_v, crows_v, xrows_v, nrows_v,
                      posb_v, negb_v, sem_idx, sem_rows):
        wid = lax.axis_index("s") * _NC + lax.axis_index("c")
        base = pl.multiple_of(wid * b_per_w, b_per_w)

        def start_idx(cb, p):
            b0 = pl.multiple_of(base + cb * CB, CB)
            pltpu.async_copy(center_hbm.at[pl.ds(b0, CB)], cidx_v[p],
                             sem_idx[p])
            pltpu.async_copy(context_hbm.at[pl.ds(b0, CB)], xidx_v[p],
                             sem_idx[p])
            pltpu.async_copy(
                neg_hbm.at[pl.ds(pl.multiple_of(b0 * NNEG, NIDX), NIDX)],
                nidx_v[p], sem_idx[p])

        def wait_idx(p):
            pltpu.make_async_copy(center_hbm.at[pl.ds(0, CB)], cidx_v[p],
                                  sem_idx[p]).wait()
            pltpu.make_async_copy(context_hbm.at[pl.ds(0, CB)], xidx_v[p],
                                  sem_idx[p]).wait()
            pltpu.make_async_copy(neg_hbm.at[pl.ds(0, NIDX)], nidx_v[p],
                                  sem_idx[p]).wait()

        def remap_idx(p):
            h = _CT // 2
            for buf, n in ((cidx_v[p], CB), (xidx_v[p], CB),
                           (nidx_v[p], NIDX)):
                for k in range(n // _L):
                    v = buf[pl.ds(k * _L, _L)]
                    buf[pl.ds(k * _L, _L)] = ((v // _CT) * _CT
                                              + (v % h) * 2 + (v // h) % 2)

        def start_rows(p):
            pltpu.async_copy(wc_hbm.at[cidx_v[p]], crows_v[p], sem_rows[p])
            pltpu.async_copy(wx_hbm.at[xidx_v[p]], xrows_v[p], sem_rows[p])
            for j in range(NSTREAM):
                pltpu.async_copy(
                    wx_hbm.at[nidx_v[p].at[pl.ds(j * 128, 128)]],
                    nrows_v[p].at[pl.ds(j * 128, 128), :], sem_rows[p])

        def wait_rows(p):
            pltpu.make_async_copy(wc_hbm.at[cidx_v[p]], crows_v[p],
                                  sem_rows[p]).wait()
            pltpu.make_async_copy(wx_hbm.at[xidx_v[p]], xrows_v[p],
                                  sem_rows[p]).wait()
            for j in range(NSTREAM):
                pltpu.make_async_copy(
                    wx_hbm.at[nidx_v[p].at[pl.ds(j * 128, 128)]],
                    nrows_v[p].at[pl.ds(j * 128, 128), :],
                    sem_rows[p]).wait()

        def compute(cb, p):
            crow, xrow, nrow = crows_v[p], xrows_v[p], nrows_v[p]

            lane = lax.iota(jnp.int32, _L)
            m_last = lane == (_L - 1)

            def b_body(b, carry):
                cv = [crow[b, pl.ds(k * _L, _L)] for k in range(NV)]
                xv = [xrow[b, pl.ds(k * _L, _L)] for k in range(NV)]
                pp = ((cv[0] * xv[0] + cv[1] * xv[1])
                      + (cv[2] * xv[2] + cv[3] * xv[3]))
                idxv = jnp.full((_L,), cb * CB + b, jnp.int32)
                plsc.store_scatter(posb_v, [idxv], plsc.cumsum(pp),
                                   mask=m_last)
                nb = b * NNEG
                for n in range(NNEG):
                    nv = [nrow[nb + n, pl.ds(k * _L, _L)] for k in range(NV)]
                    np_ = ((cv[0] * nv[0] + cv[1] * nv[1])
                           + (cv[2] * nv[2] + cv[3] * nv[3]))
                    plsc.store_scatter(
                        negb_v, [jnp.full((_L,), n, jnp.int32), idxv],
                        plsc.cumsum(np_), mask=m_last)
                return carry

            lax.fori_loop(0, CB, b_body, 0)

        start_idx(0, 0)
        wait_idx(0)
        remap_idx(0)
        start_rows(0)
        start_idx(1, 1)

        def half_body(h, carry):
            for p in range(2):
                cb = h * 2 + p
                q = 1 - p

                @pl.when(cb + 1 < NCH)
                def _():
                    wait_idx(q)
                    remap_idx(q)
                    start_rows(q)

                wait_rows(p)

                @pl.when(cb + 2 < NCH)
                def _():
                    start_idx(cb + 2, p)

                compute(cb, p)
            return carry

        lax.fori_loop(0, NCH // 2, half_body, 0)

        pltpu.sync_copy(posb_v, pos_hbm.at[pl.ds(base, b_per_w)])
        for n in range(NNEG):
            pltpu.sync_copy(negb_v.at[n], negt_hbm.at[n, pl.ds(base, b_per_w)])

    return scores_kernel(center, context, neg_flat, W_center, W_context)


_CT = 4096


def _transpose_block_kernel(wt_ref, out_ref):
    x = wt_ref[...]
    h = _CT // 2
    a = x[:, :h].T
    b = x[:, h:].T
    out_ref[...] = jnp.concatenate([a, b], axis=1)


def _to_row_major_linear(W):
    N, D = W.shape
    Wt = W.T
    nb = (N + _CT - 1) // _CT
    out = pl.pallas_call(
        _transpose_block_kernel,
        grid=(nb,),
        in_specs=[pl.BlockSpec((D, _CT), lambda i: (0, i))],
        out_specs=pl.BlockSpec((_CT // 2, 2 * D), lambda i: (i, 0)),
        out_shape=jax.ShapeDtypeStruct((nb * _CT // 2, 2 * D), jnp.float32),
    )(Wt)
    return out.reshape(nb * _CT, D)


def _loss_kernel(pos_ref, neg_ref, out_ref, *, inv_b):
    def logsig(x):
        return jnp.minimum(x, 0.0) - jnp.log1p(jnp.exp(-jnp.abs(x)))

    s_pos = jnp.sum(logsig(pos_ref[...]))
    s_neg = jnp.sum(logsig(-neg_ref[...]))
    out_ref[...] = jnp.broadcast_to(-(s_pos + s_neg) * inv_b, (1, 1))


def kernel(center, context, negatives, W_center, W_context):
    W_center, W_context = lax.optimization_barrier((W_center, W_context))
    W_center = _to_row_major_linear(W_center)
    W_context = _to_row_major_linear(W_context)
    B, NNEG = negatives.shape
    D = W_center.shape[1]
    pos, negt = _sc_scores(B, NNEG, D,
                           center.astype(jnp.int32),
                           context.astype(jnp.int32),
                           negatives.reshape(-1).astype(jnp.int32),
                           W_center, W_context)
    loss = pl.pallas_call(
        functools.partial(_loss_kernel, inv_b=1.0 / B),
        out_shape=jax.ShapeDtypeStruct((1, 1), jnp.float32),
    )(pos.reshape(B // 128, 128), negt.reshape(NNEG * B // 128, 128))
    return loss[0, 0]


# speedup vs baseline: 2.7918x; 1.4567x over previous
import functools

import jax
import jax.numpy as jnp
from jax import lax
from jax.experimental import pallas as pl
from jax.experimental.pallas import tpu as pltpu
from jax.experimental.pallas import tpu_sc as plsc

_NC = 2
_NS = 16
_NW = _NC * _NS
_L = 16


def _sc_scores(B, NNEG, D, center, context, neg_flat, W_table):
    b_per_w = B // _NW
    CB = 32
    NCH = b_per_w // CB
    NIDX = CB * NNEG
    NSTREAM = NIDX // 128

    mesh = plsc.VectorSubcoreMesh(core_axis_name="c", subcore_axis_name="s")

    NV = D // _L

    @functools.partial(
        pl.kernel,
        out_type=[
            jax.ShapeDtypeStruct((B,), jnp.float32),
            jax.ShapeDtypeStruct((NNEG, B), jnp.float32),
        ],
        mesh=mesh,
        scratch_types=[
            [pltpu.VMEM((CB,), jnp.int32)] * 2,
            [pltpu.VMEM((CB,), jnp.int32)] * 2,
            [pltpu.VMEM((NIDX,), jnp.int32)] * 2,
            [pltpu.VMEM((CB, D), jnp.float32)] * 2,
            [pltpu.VMEM((CB, D), jnp.float32)] * 2,
            [pltpu.VMEM((NIDX, D), jnp.float32)] * 2,
            pltpu.VMEM((B // _NW,), jnp.float32),
            pltpu.VMEM((NNEG, B // _NW), jnp.float32),
            [pltpu.SemaphoreType.DMA] * 2,
            [pltpu.SemaphoreType.DMA] * 2,
        ],
        compiler_params=pltpu.CompilerParams(needs_layout_passes=False,
                                             use_tc_tiling_on_sc=False),
    )
    def scores_kernel(center_hbm, context_hbm, neg_hbm, w_hbm,
                      pos_hbm, negt_hbm,
                      cidx_v, xidx_v, nidx_v, crows_v, xrows_v, nrows_v,
                      posb_v, negb_v, sem_idx, sem_rows):
        wid = lax.axis_index("s") * _NC + lax.axis_index("c")
        base = pl.multiple_of(wid * b_per_w, b_per_w)

        def start_idx(cb, p):
            b0 = pl.multiple_of(base + cb * CB, CB)
            pltpu.async_copy(center_hbm.at[pl.ds(b0, CB)], cidx_v[p],
                             sem_idx[p])
            pltpu.async_copy(context_hbm.at[pl.ds(b0, CB)], xidx_v[p],
                             sem_idx[p])
            pltpu.async_copy(
                neg_hbm.at[pl.ds(pl.multiple_of(b0 * NNEG, NIDX), NIDX)],
                nidx_v[p], sem_idx[p])

        def wait_idx(p):
            pltpu.make_async_copy(center_hbm.at[pl.ds(0, CB)], cidx_v[p],
                                  sem_idx[p]).wait()
            pltpu.make_async_copy(context_hbm.at[pl.ds(0, CB)], xidx_v[p],
                                  sem_idx[p]).wait()
            pltpu.make_async_copy(neg_hbm.at[pl.ds(0, NIDX)], nidx_v[p],
                                  sem_idx[p]).wait()

        def remap_idx(p):
            for buf, n, off in ((cidx_v[p], CB, 0), (xidx_v[p], CB, 1),
                                (nidx_v[p], NIDX, 1)):
                for k in range(n // _L):
                    v = buf[pl.ds(k * _L, _L)]
                    buf[pl.ds(k * _L, _L)] = v + v + off

        def start_rows(p):
            pltpu.async_copy(w_hbm.at[cidx_v[p]], crows_v[p], sem_rows[p])
            pltpu.async_copy(w_hbm.at[xidx_v[p]], xrows_v[p], sem_rows[p])
            for j in range(NSTREAM):
                pltpu.async_copy(
                    w_hbm.at[nidx_v[p].at[pl.ds(j * 128, 128)]],
                    nrows_v[p].at[pl.ds(j * 128, 128), :], sem_rows[p])

        def wait_rows(p):
            pltpu.make_async_copy(w_hbm.at[cidx_v[p]], crows_v[p],
                                  sem_rows[p]).wait()
            pltpu.make_async_copy(w_hbm.at[xidx_v[p]], xrows_v[p],
                                  sem_rows[p]).wait()
            for j in range(NSTREAM):
                pltpu.make_async_copy(
                    w_hbm.at[nidx_v[p].at[pl.ds(j * 128, 128)]],
                    nrows_v[p].at[pl.ds(j * 128, 128), :],
                    sem_rows[p]).wait()

        def compute(cb, p):
            crow, xrow, nrow = crows_v[p], xrows_v[p], nrows_v[p]

            lane = lax.iota(jnp.int32, _L)
            m_last = lane == (_L - 1)

            def b_body(b, carry):
                cv = [crow[b, pl.ds(k * _L, _L)] for k in range(NV)]
                xv = [xrow[b, pl.ds(k * _L, _L)] for k in range(NV)]
                pp = ((cv[0] * xv[0] + cv[1] * xv[1])
                      + (cv[2] * xv[2] + cv[3] * xv[3]))
                idxv = jnp.full((_L,), cb * CB + b, jnp.int32)
                plsc.store_scatter(posb_v, [idxv], plsc.cumsum(pp),
                                   mask=m_last)
                nb = b * NNEG
                for n in range(NNEG):
                    nv = [nrow[nb + n, pl.ds(k * _L, _L)] for k in range(NV)]
                    np_ = ((cv[0] * nv[0] + cv[1] * nv[1])
                           + (cv[2] * nv[2] + cv[3] * nv[3]))
                    plsc.store_scatter(
                        negb_v, [jnp.full((_L,), n, jnp.int32), idxv],
                        plsc.cumsum(np_), mask=m_last)
                return carry

            lax.fori_loop(0, CB, b_body, 0)

        start_idx(0, 0)
        wait_idx(0)
        remap_idx(0)
        start_rows(0)
        start_idx(1, 1)

        def half_body(h, carry):
            for p in range(2):
                cb = h * 2 + p
                q = 1 - p

                @pl.when(cb + 1 < NCH)
                def _():
                    wait_idx(q)
                    remap_idx(q)
                    start_rows(q)

                wait_rows(p)

                @pl.when(cb + 2 < NCH)
                def _():
                    start_idx(cb + 2, p)

                compute(cb, p)
            return carry

        lax.fori_loop(0, NCH // 2, half_body, 0)

        pltpu.sync_copy(posb_v, pos_hbm.at[pl.ds(base, b_per_w)])
        for n in range(NNEG):
            pltpu.sync_copy(negb_v.at[n], negt_hbm.at[n, pl.ds(base, b_per_w)])

    return scores_kernel(center, context, neg_flat, W_table)


_CT = 4096


def _merge_transpose_kernel(wc_ref, wx_ref, out_ref):
    z = jnp.concatenate([wc_ref[...], wx_ref[...]], axis=0)
    out_ref[...] = z.T


def _convert_tables(Wc, Wx):
    N, D = Wc.shape
    nb = (N + _CT - 1) // _CT
    out = pl.pallas_call(
        _merge_transpose_kernel,
        grid=(nb,),
        in_specs=[pl.BlockSpec((D, _CT), lambda i: (0, i)),
                  pl.BlockSpec((D, _CT), lambda i: (0, i))],
        out_specs=pl.BlockSpec((_CT, 2 * D), lambda i: (i, 0)),
        out_shape=jax.ShapeDtypeStruct((nb * _CT, 2 * D), jnp.float32),
    )(Wc.T, Wx.T)
    return out.reshape(2 * nb * _CT, D)


def _loss_kernel(pos_ref, neg_ref, out_ref, *, inv_b):
    def logsig(x):
        return jnp.minimum(x, 0.0) - jnp.log1p(jnp.exp(-jnp.abs(x)))

    s_pos = jnp.sum(logsig(pos_ref[...]))
    s_neg = jnp.sum(logsig(-neg_ref[...]))
    out_ref[...] = jnp.broadcast_to(-(s_pos + s_neg) * inv_b, (1, 1))


def kernel(center, context, negatives, W_center, W_context):
    W_center, W_context = lax.optimization_barrier((W_center, W_context))
    W_table = _convert_tables(W_center, W_context)
    B, NNEG = negatives.shape
    D = W_center.shape[1]
    pos, negt = _sc_scores(B, NNEG, D,
                           center.astype(jnp.int32),
                           context.astype(jnp.int32),
                           negatives.reshape(-1).astype(jnp.int32),
                           W_table)
    loss = pl.pallas_call(
        functools.partial(_loss_kernel, inv_b=1.0 / B),
        out_shape=jax.ShapeDtypeStruct((1, 1), jnp.float32),
    )(pos.reshape(B // 128, 128), negt.reshape(NNEG * B // 128, 128))
    return loss[0, 0]


# speedup vs baseline: 3.0812x; 1.1036x over previous
import functools

import jax
import jax.numpy as jnp
from jax import lax
from jax.experimental import pallas as pl
from jax.experimental.pallas import tpu as pltpu
from jax.experimental.pallas import tpu_sc as plsc

_NC = 2
_NS = 16
_NW = _NC * _NS
_L = 16


def _sc_scores(B, NNEG, D, center, context, neg_flat, W_table):
    b_per_w = B // _NW
    CB = 32
    NCH = b_per_w // CB
    NIDX = CB * NNEG
    NSTREAM = NIDX // 128

    mesh = plsc.VectorSubcoreMesh(core_axis_name="c", subcore_axis_name="s")

    NV = D // _L

    @functools.partial(
        pl.kernel,
        out_type=[
            jax.ShapeDtypeStruct((B,), jnp.float32),
            jax.ShapeDtypeStruct((NNEG, B), jnp.float32),
        ],
        mesh=mesh,
        scratch_types=[
            [pltpu.VMEM((CB,), jnp.int32)] * 2,
            [pltpu.VMEM((CB,), jnp.int32)] * 2,
            [pltpu.VMEM((NIDX,), jnp.int32)] * 2,
            [pltpu.VMEM((CB, D), jnp.float32)] * 2,
            [pltpu.VMEM((CB, D), jnp.float32)] * 2,
            [pltpu.VMEM((NIDX, D), jnp.float32)] * 2,
            pltpu.VMEM((B // _NW,), jnp.float32),
            pltpu.VMEM((NNEG, B // _NW), jnp.float32),
            [pltpu.SemaphoreType.DMA] * 2,
            [pltpu.SemaphoreType.DMA] * 2,
        ],
        compiler_params=pltpu.CompilerParams(needs_layout_passes=False,
                                             use_tc_tiling_on_sc=False),
    )
    def scores_kernel(center_hbm, context_hbm, neg_hbm, w_hbm,
                      pos_hbm, negt_hbm,
                      cidx_v, xidx_v, nidx_v, crows_v, xrows_v, nrows_v,
                      posb_v, negb_v, sem_idx, sem_rows):
        wid = lax.axis_index("s") * _NC + lax.axis_index("c")
        base = pl.multiple_of(wid * b_per_w, b_per_w)

        def start_idx(cb, p):
            b0 = pl.multiple_of(base + cb * CB, CB)
            pltpu.async_copy(center_hbm.at[pl.ds(b0, CB)], cidx_v[p],
                             sem_idx[p])
            pltpu.async_copy(context_hbm.at[pl.ds(b0, CB)], xidx_v[p],
                             sem_idx[p])
            pltpu.async_copy(
                neg_hbm.at[pl.ds(pl.multiple_of(b0 * NNEG, NIDX), NIDX)],
                nidx_v[p], sem_idx[p])

        def wait_idx(p):
            pltpu.make_async_copy(center_hbm.at[pl.ds(0, CB)], cidx_v[p],
                                  sem_idx[p]).wait()
            pltpu.make_async_copy(context_hbm.at[pl.ds(0, CB)], xidx_v[p],
                                  sem_idx[p]).wait()
            pltpu.make_async_copy(neg_hbm.at[pl.ds(0, NIDX)], nidx_v[p],
                                  sem_idx[p]).wait()

        def remap_idx(p):
            for buf, n, off in ((cidx_v[p], CB, 0), (xidx_v[p], CB, 1),
                                (nidx_v[p], NIDX, 1)):
                for k in range(n // _L):
                    v = buf[pl.ds(k * _L, _L)]
                    buf[pl.ds(k * _L, _L)] = v + v + off

        def start_rows(p):
            pltpu.async_copy(w_hbm.at[cidx_v[p]], crows_v[p], sem_rows[p])
            pltpu.async_copy(w_hbm.at[xidx_v[p]], xrows_v[p], sem_rows[p])
            for j in range(NSTREAM):
                pltpu.async_copy(
                    w_hbm.at[nidx_v[p].at[pl.ds(j * 128, 128)]],
                    nrows_v[p].at[pl.ds(j * 128, 128), :], sem_rows[p])

        def wait_rows(p):
            pltpu.make_async_copy(w_hbm.at[cidx_v[p]], crows_v[p],
                                  sem_rows[p]).wait()
            pltpu.make_async_copy(w_hbm.at[xidx_v[p]], xrows_v[p],
                                  sem_rows[p]).wait()
            for j in range(NSTREAM):
                pltpu.make_async_copy(
                    w_hbm.at[nidx_v[p].at[pl.ds(j * 128, 128)]],
                    nrows_v[p].at[pl.ds(j * 128, 128), :],
                    sem_rows[p]).wait()

        def compute(cb, p):
            crow, xrow, nrow = crows_v[p], xrows_v[p], nrows_v[p]

            lane = lax.iota(jnp.int32, _L)
            m_last = lane == (_L - 1)

            def b_body(b, carry):
                cv = [crow[b, pl.ds(k * _L, _L)] for k in range(NV)]
                xv = [xrow[b, pl.ds(k * _L, _L)] for k in range(NV)]
                pp = ((cv[0] * xv[0] + cv[1] * xv[1])
                      + (cv[2] * xv[2] + cv[3] * xv[3]))
                idxv = jnp.full((_L,), cb * CB + b, jnp.int32)
                plsc.store_scatter(posb_v, [idxv], plsc.cumsum(pp),
                                   mask=m_last)
                nb = b * NNEG
                for n in range(NNEG):
                    nv = [nrow[nb + n, pl.ds(k * _L, _L)] for k in range(NV)]
                    np_ = ((cv[0] * nv[0] + cv[1] * nv[1])
                           + (cv[2] * nv[2] + cv[3] * nv[3]))
                    plsc.store_scatter(
                        negb_v, [jnp.full((_L,), n, jnp.int32), idxv],
                        plsc.cumsum(np_), mask=m_last)
                return carry

            lax.fori_loop(0, CB, b_body, 0)

        start_idx(0, 0)
        wait_idx(0)
        remap_idx(0)
        start_rows(0)
        start_idx(1, 1)

        def half_body(h, carry):
            for p in range(2):
                cb = h * 2 + p
                q = 1 - p

                @pl.when(cb + 1 < NCH)
                def _():
                    wait_idx(q)
                    remap_idx(q)
                    start_rows(q)

                wait_rows(p)

                @pl.when(cb + 2 < NCH)
                def _():
                    start_idx(cb + 2, p)

                compute(cb, p)
            return carry

        lax.fori_loop(0, NCH // 2, half_body, 0)

        pltpu.sync_copy(posb_v, pos_hbm.at[pl.ds(base, b_per_w)])
        for n in range(NNEG):
            pltpu.sync_copy(negb_v.at[n], negt_hbm.at[n, pl.ds(base, b_per_w)])

    return scores_kernel(center, context, neg_flat, W_table)


_CT = 8192


def _merge_transpose_kernel(wc_ref, wx_ref, out_ref):
    z = jnp.concatenate([wc_ref[...], wx_ref[...]], axis=0)
    out_ref[...] = z.T


def _convert_tables(Wc, Wx):
    N, D = Wc.shape
    nb = (N + _CT - 1) // _CT
    out = pl.pallas_call(
        _merge_transpose_kernel,
        grid=(nb,),
        in_specs=[pl.BlockSpec((D, _CT), lambda i: (0, i)),
                  pl.BlockSpec((D, _CT), lambda i: (0, i))],
        out_specs=pl.BlockSpec((_CT, 2 * D), lambda i: (i, 0)),
        out_shape=jax.ShapeDtypeStruct((nb * _CT, 2 * D), jnp.float32),
    )(Wc.T, Wx.T)
    return out.reshape(2 * nb * _CT, D)


def _loss_kernel(pos_ref, neg_ref, out_ref, *, inv_b):
    def logsig(x):
        return jnp.minimum(x, 0.0) - jnp.log1p(jnp.exp(-jnp.abs(x)))

    s_pos = jnp.sum(logsig(pos_ref[...]))
    s_neg = jnp.sum(logsig(-neg_ref[...]))
    out_ref[...] = jnp.broadcast_to(-(s_pos + s_neg) * inv_b, (1, 1))


def kernel(center, context, negatives, W_center, W_context):
    W_center, W_context = lax.optimization_barrier((W_center, W_context))
    W_table = _convert_tables(W_center, W_context)
    B, NNEG = negatives.shape
    D = W_center.shape[1]
    pos, negt = _sc_scores(B, NNEG, D,
                           center.astype(jnp.int32),
                           context.astype(jnp.int32),
                           negatives.reshape(-1).astype(jnp.int32),
                           W_table)
    loss = pl.pallas_call(
        functools.partial(_loss_kernel, inv_b=1.0 / B),
        out_shape=jax.ShapeDtypeStruct((1, 1), jnp.float32),
    )(pos.reshape(B // 128, 128), negt.reshape(NNEG * B // 128, 128))
    return loss[0, 0]


# speedup vs baseline: 3.1236x; 1.0138x over previous
import functools

import jax
import jax.numpy as jnp
from jax import lax
from jax.experimental import pallas as pl
from jax.experimental.pallas import tpu as pltpu
from jax.experimental.pallas import tpu_sc as plsc

_NC = 2
_NS = 16
_NW = _NC * _NS
_L = 16


def _sc_scores(B, NNEG, D, center, context, neg_flat, W_table):
    b_per_w = B // _NW
    CB = 32
    NCH = b_per_w // CB
    NIDX = CB * NNEG
    NSTREAM = NIDX // 128

    mesh = plsc.VectorSubcoreMesh(core_axis_name="c", subcore_axis_name="s")

    NV = D // _L

    @functools.partial(
        pl.kernel,
        out_type=[
            jax.ShapeDtypeStruct((B,), jnp.float32),
            jax.ShapeDtypeStruct((NNEG, B), jnp.float32),
        ],
        mesh=mesh,
        scratch_types=[
            [pltpu.VMEM((CB,), jnp.int32)] * 2,
            [pltpu.VMEM((CB,), jnp.int32)] * 2,
            [pltpu.VMEM((NIDX,), jnp.int32)] * 2,
            [pltpu.VMEM((CB, D), jnp.float32)] * 2,
            [pltpu.VMEM((CB, D), jnp.float32)] * 2,
            [pltpu.VMEM((NIDX, D), jnp.float32)] * 2,
            pltpu.VMEM((B // _NW,), jnp.float32),
            pltpu.VMEM((NNEG, B // _NW), jnp.float32),
            [pltpu.SemaphoreType.DMA] * 2,
            [pltpu.SemaphoreType.DMA] * 2,
        ],
        compiler_params=pltpu.CompilerParams(needs_layout_passes=False,
                                             use_tc_tiling_on_sc=False),
    )
    def scores_kernel(center_hbm, context_hbm, neg_hbm, w_hbm,
                      pos_hbm, negt_hbm,
                      cidx_v, xidx_v, nidx_v, crows_v, xrows_v, nrows_v,
                      posb_v, negb_v, sem_idx, sem_rows):
        wid = lax.axis_index("s") * _NC + lax.axis_index("c")
        base = pl.multiple_of(wid * b_per_w, b_per_w)

        def start_idx(cb, p):
            b0 = pl.multiple_of(base + cb * CB, CB)
            pltpu.async_copy(center_hbm.at[pl.ds(b0, CB)], cidx_v[p],
                             sem_idx[p])
            pltpu.async_copy(context_hbm.at[pl.ds(b0, CB)], xidx_v[p],
                             sem_idx[p])
            pltpu.async_copy(
                neg_hbm.at[pl.ds(pl.multiple_of(b0 * NNEG, NIDX), NIDX)],
                nidx_v[p], sem_idx[p])

        def wait_idx(p):
            pltpu.make_async_copy(center_hbm.at[pl.ds(0, CB)], cidx_v[p],
                                  sem_idx[p]).wait()
            pltpu.make_async_copy(context_hbm.at[pl.ds(0, CB)], xidx_v[p],
                                  sem_idx[p]).wait()
            pltpu.make_async_copy(neg_hbm.at[pl.ds(0, NIDX)], nidx_v[p],
                                  sem_idx[p]).wait()

        def remap_idx(p):
            for buf, n, off in ((cidx_v[p], CB, 0), (xidx_v[p], CB, 1),
                                (nidx_v[p], NIDX, 1)):
                for k in range(n // _L):
                    v = buf[pl.ds(k * _L, _L)]
                    buf[pl.ds(k * _L, _L)] = v + v + off

        def start_rows(p):
            pltpu.async_copy(w_hbm.at[cidx_v[p]], crows_v[p], sem_rows[p])
            pltpu.async_copy(w_hbm.at[xidx_v[p]], xrows_v[p], sem_rows[p])
            for j in range(NSTREAM):
                pltpu.async_copy(
                    w_hbm.at[nidx_v[p].at[pl.ds(j * 128, 128)]],
                    nrows_v[p].at[pl.ds(j * 128, 128), :], sem_rows[p])

        def wait_rows(p):
            pltpu.make_async_copy(w_hbm.at[cidx_v[p]], crows_v[p],
                                  sem_rows[p]).wait()
            pltpu.make_async_copy(w_hbm.at[xidx_v[p]], xrows_v[p],
                                  sem_rows[p]).wait()
            for j in range(NSTREAM):
                pltpu.make_async_copy(
                    w_hbm.at[nidx_v[p].at[pl.ds(j * 128, 128)]],
                    nrows_v[p].at[pl.ds(j * 128, 128), :],
                    sem_rows[p]).wait()

        def compute(cb, p):
            crow, xrow, nrow = crows_v[p], xrows_v[p], nrows_v[p]

            lane = lax.iota(jnp.int32, _L)
            m_last = lane == (_L - 1)

            def b_body(b, carry):
                cv = [crow[b, pl.ds(k * _L, _L)] for k in range(NV)]
                xv = [xrow[b, pl.ds(k * _L, _L)] for k in range(NV)]
                pp = ((cv[0] * xv[0] + cv[1] * xv[1])
                      + (cv[2] * xv[2] + cv[3] * xv[3]))
                idxv = jnp.full((_L,), cb * CB + b, jnp.int32)
                plsc.store_scatter(posb_v, [idxv], plsc.cumsum(pp),
                                   mask=m_last)
                nb = b * NNEG
                for n in range(NNEG):
                    nv = [nrow[nb + n, pl.ds(k * _L, _L)] for k in range(NV)]
                    np_ = ((cv[0] * nv[0] + cv[1] * nv[1])
                           + (cv[2] * nv[2] + cv[3] * nv[3]))
                    plsc.store_scatter(
                        negb_v, [jnp.full((_L,), n, jnp.int32), idxv],
                        plsc.cumsum(np_), mask=m_last)
                return carry

            lax.fori_loop(0, CB, b_body, 0)

        start_idx(0, 0)
        wait_idx(0)
        remap_idx(0)
        start_rows(0)
        start_idx(1, 1)

        def half_body(h, carry):
            for p in range(2):
                cb = h * 2 + p
                q = 1 - p

                @pl.when(cb + 1 < NCH)
                def _():
                    wait_idx(q)
                    remap_idx(q)
                    start_rows(q)

                wait_rows(p)

                @pl.when(cb + 2 < NCH)
                def _():
                    start_idx(cb + 2, p)

                compute(cb, p)
            return carry

        lax.fori_loop(0, NCH // 2, half_body, 0)

        pltpu.sync_copy(posb_v, pos_hbm.at[pl.ds(base, b_per_w)])
        for n in range(NNEG):
            pltpu.sync_copy(negb_v.at[n], negt_hbm.at[n, pl.ds(base, b_per_w)])

    return scores_kernel(center, context, neg_flat, W_table)


_CT = 16384


def _merge_transpose_kernel(wc_ref, wx_ref, out_ref):
    z = jnp.concatenate([wc_ref[...], wx_ref[...]], axis=0)
    out_ref[...] = z.T


def _convert_tables(Wc, Wx):
    N, D = Wc.shape
    nb = (N + _CT - 1) // _CT
    out = pl.pallas_call(
        _merge_transpose_kernel,
        grid=(nb,),
        in_specs=[pl.BlockSpec((D, _CT), lambda i: (0, i)),
                  pl.BlockSpec((D, _CT), lambda i: (0, i))],
        out_specs=pl.BlockSpec((_CT, 2 * D), lambda i: (i, 0)),
        out_shape=jax.ShapeDtypeStruct((nb * _CT, 2 * D), jnp.float32),
    )(Wc.T, Wx.T)
    return out.reshape(2 * nb * _CT, D)


def _loss_kernel(pos_ref, neg_ref, out_ref, *, inv_b):
    def logsig(x):
        return jnp.minimum(x, 0.0) - jnp.log1p(jnp.exp(-jnp.abs(x)))

    s_pos = jnp.sum(logsig(pos_ref[...]))
    s_neg = jnp.sum(logsig(-neg_ref[...]))
    out_ref[...] = jnp.broadcast_to(-(s_pos + s_neg) * inv_b, (1, 1))


def kernel(center, context, negatives, W_center, W_context):
    W_center, W_context = lax.optimization_barrier((W_center, W_context))
    W_table = _convert_tables(W_center, W_context)
    B, NNEG = negatives.shape
    D = W_center.shape[1]
    pos, negt = _sc_scores(B, NNEG, D,
                           center.astype(jnp.int32),
                           context.astype(jnp.int32),
                           negatives.reshape(-1).astype(jnp.int32),
                           W_table)
    loss = pl.pallas_call(
        functools.partial(_loss_kernel, inv_b=1.0 / B),
        out_shape=jax.ShapeDtypeStruct((1, 1), jnp.float32),
    )(pos.reshape(B // 128, 128), negt.reshape(NNEG * B // 128, 128))
    return loss[0, 0]
